# Initial kernel scaffold; baseline (speedup 1.0000x reference)
import jax
import jax.numpy as jnp
from jax.experimental import pallas as pl


def kernel(x, theta, edge_index):
    raise NotImplementedError("write your pallas kernel here")


# speedup vs baseline: 179.7736x; 179.7736x over previous
import jax
import jax.numpy as jnp
from jax import lax
from jax.experimental import pallas as pl
from jax.experimental.pallas import tpu as pltpu
from jax.experimental.pallas import tpu_sc as plsc

_V0 = 1.0
_W0 = 1.0
_N = 100000
_E = 6400000

_LANE = 128
_ROWS = 782
_NP = _ROWS * _LANE

_NC = 2
_NS = 16
_K = 10
_CE = _K * _LANE

_CH_TILES = 8
_CH_EDGES = _E // _CH_TILES
_CH_CHUNKS = _CH_EDGES // _CE
_CNT_TILES = 25
_CNT_EDGES = _E // _CNT_TILES
_CNT_CHUNKS = _CNT_EDGES // _CE


def _al(v):
    return pl.multiple_of(v, 8)


def _pre_body(x0_ref, x1_ref, f_ref):
    x0 = x0_ref[...]
    x1 = x1_ref[...]
    f_ref[0] = jnp.cos(x0)
    f_ref[1] = jnp.cos(x1)
    f_ref[2] = jnp.sin(x0)
    f_ref[3] = jnp.sin(x1)


_pre_call = pl.pallas_call(
    _pre_body,
    out_shape=jax.ShapeDtypeStruct((4, _ROWS, _LANE), jnp.float32),
)


def _epi_body(p_ref, t_ref, v0_ref, v1_ref, w_ref):
    tv = t_ref[...]
    ct = jnp.cos(tv)
    st = jnp.sin(tv)
    sc0 = p_ref[0, 0]
    sc1 = p_ref[0, 1]
    ss0 = p_ref[1, 0]
    ss1 = p_ref[1, 1]
    cnt = p_ref[0, 2] + p_ref[1, 2]
    m0 = ct * sc0 + st * ss0
    m1 = ct * sc1 + st * ss1
    m2 = ct * ss0 - st * sc0
    m3 = ct * ss1 - st * sc1
    inv = 1.0 / jnp.maximum(cnt, 1.0)
    a0 = m0 * inv
    a1 = m1 * inv
    a2 = m2 * inv
    a3 = m3 * inv
    norm = jnp.sqrt(a0 * a0 + a1 * a1 + a2 * a2 + a3 * a3)
    w_ref[...] = _W0 * (a1 / jnp.maximum(norm, 1e-12))
    v0_ref[...] = _V0 * ct
    v1_ref[...] = _V0 * st


_epi_call = pl.pallas_call(
    _epi_body,
    out_shape=(
        jax.ShapeDtypeStruct((_ROWS, _LANE), jnp.float32),
        jax.ShapeDtypeStruct((_ROWS, _LANE), jnp.float32),
        jax.ShapeDtypeStruct((_ROWS, _LANE), jnp.float32),
    ),
)


def _sc_body(ftab_hbm, src_hbm, dst_hbm, zero_hbm, out_hbm,
             ftab_v, src_v, dst_v, idx_v, val_v, ones_v, acc,
             sem_a, sem_b, sem_add):
    c = lax.axis_index("c")
    s = lax.axis_index("s")
    wid = c * _NS + s
    ch = 2 * c + s // _CH_TILES
    slot = s // _CH_TILES
    rank = s % _CH_TILES
    sems = (sem_a, sem_b)

    for i in range(_LANE // 16):
        ones_v[pl.ds(i * 16, 16)] = jnp.full((16,), 1.0, jnp.float32)

    pltpu.sync_copy(ftab_hbm.at[pl.ds(_al(ch * _NP), _NP)], ftab_v)

    @pl.when(s == 0)
    def _():
        pltpu.sync_copy(zero_hbm, acc)

    plsc.subcore_barrier()

    choff = slot * _NP

    def process_rows(b, gather, off):
        handles = []
        for j in range(_K):
            for l in range(_LANE // 16):
                sl = pl.ds(j * _LANE + l * 16, 16)
                sl2 = pl.ds(l * 16, 16)
                idx_v[j, sl2] = dst_v[b, sl] + off
                if gather:
                    val_v[j, sl2] = plsc.load_gather(ftab_v, [src_v[b, sl]])
            vsrc = val_v.at[j] if gather else ones_v
            handles.append(
                pltpu.async_copy(vsrc, acc.at[idx_v.at[j]], sem_add, add=True))
        for h in handles:
            h.wait()

    e0 = rank * _CH_EDGES
    for b in range(2):
        pltpu.async_copy(src_hbm.at[pl.ds(e0 + b * _CE, _CE)],
                         src_v.at[b], sems[b])
        pltpu.async_copy(dst_hbm.at[pl.ds(e0 + b * _CE, _CE)],
                         dst_v.at[b], sems[b])

    last_ch = _CH_CHUNKS - 1

    @pl.loop(0, _CH_CHUNKS, step=2)
    def _ch_chunks(g):
        for b in range(2):
            gb = g + b

            @pl.when(gb <= last_ch)
            def _():
                rb = _al(e0 + gb * _CE)
                pltpu.make_async_copy(src_hbm.at[pl.ds(rb, _CE)],
                                      src_v.at[b], sems[b]).wait()
                pltpu.make_async_copy(dst_hbm.at[pl.ds(rb, _CE)],
                                      dst_v.at[b], sems[b]).wait()
                process_rows(b, True, choff)

                @pl.when(gb + 2 <= last_ch)
                def _():
                    nb = _al(e0 + (gb + 2) * _CE)
                    pltpu.async_copy(src_hbm.at[pl.ds(nb, _CE)],
                                     src_v.at[b], sems[b])
                    pltpu.async_copy(dst_hbm.at[pl.ds(nb, _CE)],
                                     dst_v.at[b], sems[b])

    @pl.when(wid < _CNT_TILES)
    def _():
        c0 = wid * _CNT_EDGES
        for b in range(2):
            pltpu.async_copy(dst_hbm.at[pl.ds(_al(c0 + b * _CE), _CE)],
                             dst_v.at[b], sems[b])

        last_cnt = _CNT_CHUNKS - 1

        @pl.loop(0, _CNT_CHUNKS, step=2)
        def _cnt_chunks(g):
            for b in range(2):
                gb = g + b

                @pl.when(gb <= last_cnt)
                def _():
                    rb = _al(c0 + gb * _CE)
                    pltpu.make_async_copy(dst_hbm.at[pl.ds(rb, _CE)],
                                          dst_v.at[b], sems[b]).wait()
                    process_rows(b, False, 2 * _NP)

                    @pl.when(gb + 2 <= last_cnt)
                    def _():
                        nb = _al(c0 + (gb + 2) * _CE)
                        pltpu.async_copy(dst_hbm.at[pl.ds(nb, _CE)],
                                         dst_v.at[b], sems[b])

    plsc.subcore_barrier()

    @pl.when(s == 0)
    def _():
        pltpu.sync_copy(acc, out_hbm.at[pl.ds(_al(c * 3 * _NP), 3 * _NP)])


_sc_call = pl.kernel(
    _sc_body,
    out_type=jax.ShapeDtypeStruct((_NC * 3 * _NP,), jnp.float32),
    mesh=plsc.VectorSubcoreMesh(core_axis_name="c", subcore_axis_name="s"),
    compiler_params=pltpu.CompilerParams(needs_layout_passes=False),
    scratch_types=[
        pltpu.VMEM((_NP,), jnp.float32),
        pltpu.VMEM((2, _CE), jnp.int32),
        pltpu.VMEM((2, _CE), jnp.int32),
        pltpu.VMEM((_K, _LANE), jnp.int32),
        pltpu.VMEM((_K, _LANE), jnp.float32),
        pltpu.VMEM((_LANE,), jnp.float32),
        pltpu.VMEM_SHARED((3 * _NP,), jnp.float32),
        pltpu.SemaphoreType.DMA,
        pltpu.SemaphoreType.DMA,
        pltpu.SemaphoreType.DMA,
    ],
)


@jax.jit
def kernel(x, theta, edge_index):
    x = x.astype(jnp.float32)
    theta = theta.astype(jnp.float32)
    pad = _NP - _N
    x0 = jnp.pad(x[:, 0], (0, pad)).reshape(_ROWS, _LANE)
    x1 = jnp.pad(x[:, 1], (0, pad)).reshape(_ROWS, _LANE)
    tp = jnp.pad(theta[:, 0], (0, pad)).reshape(_ROWS, _LANE)

    ftab = _pre_call(x0, x1).reshape(4 * _NP)
    src = edge_index[0]
    dst = edge_index[1]
    zeros = jnp.zeros((3 * _NP,), jnp.float32)

    part = _sc_call(ftab, src, dst, zeros)
    pp = part.reshape(_NC, 3, _ROWS, _LANE)

    v0, v1, w = _epi_call(pp, tp)
    v = jnp.stack([v0.reshape(-1)[:_N], v1.reshape(-1)[:_N]], axis=-1)
    wq = w.reshape(-1)[:_N][:, None]
    return (v, wq)


# speedup vs baseline: 216.6465x; 1.2051x over previous
import jax
import jax.numpy as jnp
from jax import lax
from jax.experimental import pallas as pl
from jax.experimental.pallas import tpu as pltpu
from jax.experimental.pallas import tpu_sc as plsc

_V0 = 1.0
_W0 = 1.0
_N = 100000
_E = 6400000

_LANE = 128
_ROWS = 782
_NP = _ROWS * _LANE

_NC = 2
_NS = 16
_K = 10
_CE = _K * _LANE

_CH_TILES = 8
_CH_EDGES = _E // _CH_TILES
_CH_CHUNKS = _CH_EDGES // _CE


def _al(v):
    return pl.multiple_of(v, 8)


def _pre_body(x0_ref, x1_ref, f_ref):
    x0 = x0_ref[...]
    x1 = x1_ref[...]
    f_ref[0] = jnp.cos(x0)
    f_ref[1] = jnp.cos(x1)
    f_ref[2] = jnp.sin(x0)
    f_ref[3] = jnp.sin(x1)


_pre_call = pl.pallas_call(
    _pre_body,
    out_shape=jax.ShapeDtypeStruct((4, _ROWS, _LANE), jnp.float32),
)


def _epi_body(p_ref, t_ref, v0_ref, v1_ref, w_ref):
    tv = t_ref[...]
    ct = jnp.cos(tv)
    st = jnp.sin(tv)
    sc0 = p_ref[0, 0]
    sc1 = p_ref[0, 1]
    ss0 = p_ref[1, 0]
    ss1 = p_ref[1, 1]
    m0 = ct * sc0 + st * ss0
    m1 = ct * sc1 + st * ss1
    m2 = ct * ss0 - st * sc0
    m3 = ct * ss1 - st * sc1
    norm = jnp.sqrt(m0 * m0 + m1 * m1 + m2 * m2 + m3 * m3)
    w_ref[...] = _W0 * (m1 / jnp.maximum(norm, 1e-12))
    v0_ref[...] = _V0 * ct
    v1_ref[...] = _V0 * st


_epi_call = pl.pallas_call(
    _epi_body,
    out_shape=(
        jax.ShapeDtypeStruct((_ROWS, _LANE), jnp.float32),
        jax.ShapeDtypeStruct((_ROWS, _LANE), jnp.float32),
        jax.ShapeDtypeStruct((_ROWS, _LANE), jnp.float32),
    ),
)


def _sc_body(ftab_hbm, src_hbm, dst_hbm, zero_hbm, out_hbm,
             ftab_v, src_v, dst_v, idx_v, val_v, acc,
             sem_a, sem_b, sem_add):
    c = lax.axis_index("c")
    s = lax.axis_index("s")
    wid = c * _NS + s
    ch = 2 * c + s // _CH_TILES
    slot = s // _CH_TILES
    rank = s % _CH_TILES
    sems = (sem_a, sem_b)

    pltpu.sync_copy(ftab_hbm.at[pl.ds(_al(ch * _NP), _NP)], ftab_v)

    @pl.when(s == 0)
    def _():
        pltpu.sync_copy(zero_hbm, acc)

    plsc.subcore_barrier()

    choff = slot * _NP

    def process_rows(b, off):
        handles = []
        for j in range(_K):
            for l in range(_LANE // 16):
                sl = pl.ds(j * _LANE + l * 16, 16)
                sl2 = pl.ds(l * 16, 16)
                idx_v[j, sl2] = dst_v[b, sl] + off
                val_v[j, sl2] = plsc.load_gather(ftab_v, [src_v[b, sl]])
            handles.append(
                pltpu.async_copy(val_v.at[j], acc.at[idx_v.at[j]],
                                 sem_add, add=True))
        for h in handles:
            h.wait()

    e0 = rank * _CH_EDGES
    for b in range(2):
        pltpu.async_copy(src_hbm.at[pl.ds(e0 + b * _CE, _CE)],
                         src_v.at[b], sems[b])
        pltpu.async_copy(dst_hbm.at[pl.ds(e0 + b * _CE, _CE)],
                         dst_v.at[b], sems[b])

    last_ch = _CH_CHUNKS - 1

    @pl.loop(0, _CH_CHUNKS, step=2)
    def _ch_chunks(g):
        for b in range(2):
            gb = g + b

            @pl.when(gb <= last_ch)
            def _():
                rb = _al(e0 + gb * _CE)
                pltpu.make_async_copy(src_hbm.at[pl.ds(rb, _CE)],
                                      src_v.at[b], sems[b]).wait()
                pltpu.make_async_copy(dst_hbm.at[pl.ds(rb, _CE)],
                                      dst_v.at[b], sems[b]).wait()
                process_rows(b, choff)

                @pl.when(gb + 2 <= last_ch)
                def _():
                    nb = _al(e0 + (gb + 2) * _CE)
                    pltpu.async_copy(src_hbm.at[pl.ds(nb, _CE)],
                                     src_v.at[b], sems[b])
                    pltpu.async_copy(dst_hbm.at[pl.ds(nb, _CE)],
                                     dst_v.at[b], sems[b])

    plsc.subcore_barrier()

    @pl.when(s == 0)
    def _():
        pltpu.sync_copy(acc, out_hbm.at[pl.ds(_al(c * 2 * _NP), 2 * _NP)])


_sc_call = pl.kernel(
    _sc_body,
    out_type=jax.ShapeDtypeStruct((_NC * 2 * _NP,), jnp.float32),
    mesh=plsc.VectorSubcoreMesh(core_axis_name="c", subcore_axis_name="s"),
    compiler_params=pltpu.CompilerParams(needs_layout_passes=False),
    scratch_types=[
        pltpu.VMEM((_NP,), jnp.float32),
        pltpu.VMEM((2, _CE), jnp.int32),
        pltpu.VMEM((2, _CE), jnp.int32),
        pltpu.VMEM((_K, _LANE), jnp.int32),
        pltpu.VMEM((_K, _LANE), jnp.float32),
        pltpu.VMEM_SHARED((2 * _NP,), jnp.float32),
        pltpu.SemaphoreType.DMA,
        pltpu.SemaphoreType.DMA,
        pltpu.SemaphoreType.DMA,
    ],
)


@jax.jit
def kernel(x, theta, edge_index):
    x = x.astype(jnp.float32)
    theta = theta.astype(jnp.float32)
    pad = _NP - _N
    x0 = jnp.pad(x[:, 0], (0, pad)).reshape(_ROWS, _LANE)
    x1 = jnp.pad(x[:, 1], (0, pad)).reshape(_ROWS, _LANE)
    tp = jnp.pad(theta[:, 0], (0, pad)).reshape(_ROWS, _LANE)

    ftab = _pre_call(x0, x1).reshape(4 * _NP)
    src = edge_index[0]
    dst = edge_index[1]
    zeros = jnp.zeros((2 * _NP,), jnp.float32)

    part = _sc_call(ftab, src, dst, zeros)
    pp = part.reshape(_NC, 2, _ROWS, _LANE)

    v0, v1, w = _epi_call(pp, tp)
    v = jnp.stack([v0.reshape(-1)[:_N], v1.reshape(-1)[:_N]], axis=-1)
    wq = w.reshape(-1)[:_N][:, None]
    return (v, wq)


# speedup vs baseline: 229.6911x; 1.0602x over previous
import jax
import jax.numpy as jnp
from jax import lax
from jax.experimental import pallas as pl
from jax.experimental.pallas import tpu as pltpu
from jax.experimental.pallas import tpu_sc as plsc

_V0 = 1.0
_W0 = 1.0
_N = 100000
_E = 6400000

_LANE = 128
_ROWS = 782
_NP = _ROWS * _LANE

_NC = 2
_NS = 16
_K = 10
_CE = _K * _LANE

_CH_TILES = 8
_CH_EDGES = _E // _CH_TILES
_CH_CHUNKS = _CH_EDGES // _CE


def _al(v):
    return pl.multiple_of(v, 8)


def _pre_body(x0_ref, x1_ref, f_ref):
    x0 = x0_ref[...]
    x1 = x1_ref[...]
    f_ref[0] = jnp.cos(x0)
    f_ref[1] = jnp.cos(x1)
    f_ref[2] = jnp.sin(x0)
    f_ref[3] = jnp.sin(x1)


_pre_call = pl.pallas_call(
    _pre_body,
    out_shape=jax.ShapeDtypeStruct((4, _ROWS, _LANE), jnp.float32),
)


def _epi_body(p_ref, t_ref, v0_ref, v1_ref, w_ref):
    tv = t_ref[...]
    ct = jnp.cos(tv)
    st = jnp.sin(tv)
    sc0 = p_ref[0, 0]
    sc1 = p_ref[0, 1]
    ss0 = p_ref[1, 0]
    ss1 = p_ref[1, 1]
    m0 = ct * sc0 + st * ss0
    m1 = ct * sc1 + st * ss1
    m2 = ct * ss0 - st * sc0
    m3 = ct * ss1 - st * sc1
    norm = jnp.sqrt(m0 * m0 + m1 * m1 + m2 * m2 + m3 * m3)
    w_ref[...] = _W0 * (m1 / jnp.maximum(norm, 1e-12))
    v0_ref[...] = _V0 * ct
    v1_ref[...] = _V0 * st


_epi_call = pl.pallas_call(
    _epi_body,
    out_shape=(
        jax.ShapeDtypeStruct((_ROWS, _LANE), jnp.float32),
        jax.ShapeDtypeStruct((_ROWS, _LANE), jnp.float32),
        jax.ShapeDtypeStruct((_ROWS, _LANE), jnp.float32),
    ),
)


def _sc_body(ftab_hbm, edges_hbm, zero_hbm, out_hbm,
             ftab_v, src_v, dst_v, idx_v, val_v, acc,
             sem_a, sem_b, sem_add):
    c = lax.axis_index("c")
    s = lax.axis_index("s")
    wid = c * _NS + s
    ch = 2 * c + s // _CH_TILES
    slot = s // _CH_TILES
    rank = s % _CH_TILES
    sems = (sem_a, sem_b)

    pltpu.sync_copy(ftab_hbm.at[pl.ds(_al(ch * _NP), _NP)], ftab_v)

    @pl.when(s == 0)
    def _():
        pltpu.sync_copy(zero_hbm, acc)

    plsc.subcore_barrier()

    choff = slot * _NP

    def process_rows(b, off):
        handles = []
        for j in range(_K):
            for l in range(_LANE // 16):
                sl = pl.ds(j * _LANE + l * 16, 16)
                sl2 = pl.ds(l * 16, 16)
                idx_v[j, sl2] = dst_v[b, sl] + off
                val_v[j, sl2] = plsc.load_gather(ftab_v, [src_v[b, sl]])
            handles.append(
                pltpu.async_copy(val_v.at[j], acc.at[idx_v.at[j]],
                                 sem_add, add=True))
        for h in handles:
            h.wait()

    e0 = rank * _CH_EDGES
    for b in range(2):
        pltpu.async_copy(edges_hbm.at[pl.ds(e0 + b * _CE, _CE)],
                         src_v.at[b], sems[b])
        pltpu.async_copy(edges_hbm.at[pl.ds(_E + e0 + b * _CE, _CE)],
                         dst_v.at[b], sems[b])

    last_ch = _CH_CHUNKS - 1

    @pl.loop(0, _CH_CHUNKS, step=2)
    def _ch_chunks(g):
        for b in range(2):
            gb = g + b

            @pl.when(gb <= last_ch)
            def _():
                rb = _al(e0 + gb * _CE)
                pltpu.make_async_copy(edges_hbm.at[pl.ds(rb, _CE)],
                                      src_v.at[b], sems[b]).wait()
                pltpu.make_async_copy(edges_hbm.at[pl.ds(_E + rb, _CE)],
                                      dst_v.at[b], sems[b]).wait()
                process_rows(b, choff)

                @pl.when(gb + 2 <= last_ch)
                def _():
                    nb = _al(e0 + (gb + 2) * _CE)
                    pltpu.async_copy(edges_hbm.at[pl.ds(nb, _CE)],
                                     src_v.at[b], sems[b])
                    pltpu.async_copy(edges_hbm.at[pl.ds(_E + nb, _CE)],
                                     dst_v.at[b], sems[b])

    plsc.subcore_barrier()

    @pl.when(s == 0)
    def _():
        pltpu.sync_copy(acc, out_hbm.at[pl.ds(_al(c * 2 * _NP), 2 * _NP)])


_sc_call = pl.kernel(
    _sc_body,
    out_type=jax.ShapeDtypeStruct((_NC * 2 * _NP,), jnp.float32),
    mesh=plsc.VectorSubcoreMesh(core_axis_name="c", subcore_axis_name="s"),
    compiler_params=pltpu.CompilerParams(needs_layout_passes=False),
    scratch_types=[
        pltpu.VMEM((_NP,), jnp.float32),
        pltpu.VMEM((2, _CE), jnp.int32),
        pltpu.VMEM((2, _CE), jnp.int32),
        pltpu.VMEM((_K, _LANE), jnp.int32),
        pltpu.VMEM((_K, _LANE), jnp.float32),
        pltpu.VMEM_SHARED((2 * _NP,), jnp.float32),
        pltpu.SemaphoreType.DMA,
        pltpu.SemaphoreType.DMA,
        pltpu.SemaphoreType.DMA,
    ],
)


@jax.jit
def kernel(x, theta, edge_index):
    x = x.astype(jnp.float32)
    theta = theta.astype(jnp.float32)
    pad = _NP - _N
    x0 = jnp.pad(x[:, 0], (0, pad)).reshape(_ROWS, _LANE)
    x1 = jnp.pad(x[:, 1], (0, pad)).reshape(_ROWS, _LANE)
    tp = jnp.pad(theta[:, 0], (0, pad)).reshape(_ROWS, _LANE)

    ftab = _pre_call(x0, x1).reshape(4 * _NP)
    edges = edge_index.reshape(2 * _E)
    zeros = jnp.zeros((2 * _NP,), jnp.float32)

    part = _sc_call(ftab, edges, zeros)
    pp = part.reshape(_NC, 2, _ROWS, _LANE)

    v0, v1, w = _epi_call(pp, tp)
    v = jnp.stack([v0.reshape(-1)[:_N], v1.reshape(-1)[:_N]], axis=-1)
    wq = w.reshape(-1)[:_N][:, None]
    return (v, wq)


# speedup vs baseline: 239.4553x; 1.0425x over previous
import jax
import jax.numpy as jnp
from jax import lax
from jax.experimental import pallas as pl
from jax.experimental.pallas import tpu as pltpu
from jax.experimental.pallas import tpu_sc as plsc

_V0 = 1.0
_W0 = 1.0
_N = 100000
_E = 6400000

_LANE = 128
_ROWS = 782
_NP = _ROWS * _LANE

_NC = 2
_NS = 16
_K = 10
_CE = _K * _LANE

_CH_TILES = 8
_CH_EDGES = _E // _CH_TILES
_CH_CHUNKS = _CH_EDGES // _CE


def _al(v):
    return pl.multiple_of(v, 8)


def _pre_body(x0_ref, x1_ref, f_ref):
    x0 = x0_ref[...]
    x1 = x1_ref[...]
    f_ref[0] = jnp.cos(x0)
    f_ref[1] = jnp.cos(x1)
    f_ref[2] = jnp.sin(x0)
    f_ref[3] = jnp.sin(x1)


_pre_call = pl.pallas_call(
    _pre_body,
    out_shape=jax.ShapeDtypeStruct((4, _ROWS, _LANE), jnp.float32),
)


def _epi_body(p_ref, t_ref, v0_ref, v1_ref, w_ref):
    tv = t_ref[...]
    ct = jnp.cos(tv)
    st = jnp.sin(tv)
    sc0 = p_ref[0, 0]
    sc1 = p_ref[0, 1]
    ss0 = p_ref[1, 0]
    ss1 = p_ref[1, 1]
    m0 = ct * sc0 + st * ss0
    m1 = ct * sc1 + st * ss1
    m2 = ct * ss0 - st * sc0
    m3 = ct * ss1 - st * sc1
    norm = jnp.sqrt(m0 * m0 + m1 * m1 + m2 * m2 + m3 * m3)
    w_ref[...] = _W0 * (m1 / jnp.maximum(norm, 1e-12))
    v0_ref[...] = _V0 * ct
    v1_ref[...] = _V0 * st


_epi_call = pl.pallas_call(
    _epi_body,
    out_shape=(
        jax.ShapeDtypeStruct((_ROWS, _LANE), jnp.float32),
        jax.ShapeDtypeStruct((_ROWS, _LANE), jnp.float32),
        jax.ShapeDtypeStruct((_ROWS, _LANE), jnp.float32),
    ),
)


def _sc_body(ftab_hbm, edges_hbm, zero_hbm, out_hbm,
             ftab_v, src_v, dst_v, idx_v, val_v, acc,
             sem_a, sem_b, sem_add):
    c = lax.axis_index("c")
    s = lax.axis_index("s")
    wid = c * _NS + s
    ch = 2 * c + s // _CH_TILES
    slot = s // _CH_TILES
    rank = s % _CH_TILES
    sems = (sem_a, sem_b)

    pltpu.sync_copy(ftab_hbm.at[pl.ds(_al(ch * _NP), _NP)], ftab_v)

    @pl.when(s == 0)
    def _():
        pltpu.sync_copy(zero_hbm, acc)

    plsc.subcore_barrier()

    choff = slot * _NP

    def process_rows(b, off):
        for j in range(_K):
            for l in range(_LANE // 16):
                sl = pl.ds(j * _LANE + l * 16, 16)
                sl2 = pl.ds(l * 16, 16)
                idx_v[j, sl2] = dst_v[b, sl] + off
                val_v[j, sl2] = plsc.load_gather(ftab_v, [src_v[b, sl]])
            pltpu.async_copy(val_v.at[j], acc.at[idx_v.at[j]],
                             sem_add, add=True)

    def drain_rows():
        for j in range(_K):
            pltpu.make_async_copy(val_v.at[j], acc.at[idx_v.at[j]],
                                  sem_add).wait()

    e0 = rank * _CH_EDGES
    for b in range(2):
        pltpu.async_copy(edges_hbm.at[pl.ds(e0 + b * _CE, _CE)],
                         src_v.at[b], sems[b])
        pltpu.async_copy(edges_hbm.at[pl.ds(_E + e0 + b * _CE, _CE)],
                         dst_v.at[b], sems[b])

    last_ch = _CH_CHUNKS - 1

    @pl.loop(0, _CH_CHUNKS, step=2)
    def _ch_chunks(g):
        for b in range(2):
            gb = g + b

            @pl.when(gb <= last_ch)
            def _():
                rb = _al(e0 + gb * _CE)
                pltpu.make_async_copy(edges_hbm.at[pl.ds(rb, _CE)],
                                      src_v.at[b], sems[b]).wait()
                pltpu.make_async_copy(edges_hbm.at[pl.ds(_E + rb, _CE)],
                                      dst_v.at[b], sems[b]).wait()

                @pl.when(gb > 0)
                def _():
                    drain_rows()

                process_rows(b, choff)

                @pl.when(gb + 2 <= last_ch)
                def _():
                    nb = _al(e0 + (gb + 2) * _CE)
                    pltpu.async_copy(edges_hbm.at[pl.ds(nb, _CE)],
                                     src_v.at[b], sems[b])
                    pltpu.async_copy(edges_hbm.at[pl.ds(_E + nb, _CE)],
                                     dst_v.at[b], sems[b])

    drain_rows()
    plsc.subcore_barrier()

    @pl.when(s == 0)
    def _():
        pltpu.sync_copy(acc, out_hbm.at[pl.ds(_al(c * 2 * _NP), 2 * _NP)])


_sc_call = pl.kernel(
    _sc_body,
    out_type=jax.ShapeDtypeStruct((_NC * 2 * _NP,), jnp.float32),
    mesh=plsc.VectorSubcoreMesh(core_axis_name="c", subcore_axis_name="s"),
    compiler_params=pltpu.CompilerParams(needs_layout_passes=False),
    scratch_types=[
        pltpu.VMEM((_NP,), jnp.float32),
        pltpu.VMEM((2, _CE), jnp.int32),
        pltpu.VMEM((2, _CE), jnp.int32),
        pltpu.VMEM((_K, _LANE), jnp.int32),
        pltpu.VMEM((_K, _LANE), jnp.float32),
        pltpu.VMEM_SHARED((2 * _NP,), jnp.float32),
        pltpu.SemaphoreType.DMA,
        pltpu.SemaphoreType.DMA,
        pltpu.SemaphoreType.DMA,
    ],
)


@jax.jit
def kernel(x, theta, edge_index):
    x = x.astype(jnp.float32)
    theta = theta.astype(jnp.float32)
    pad = _NP - _N
    x0 = jnp.pad(x[:, 0], (0, pad)).reshape(_ROWS, _LANE)
    x1 = jnp.pad(x[:, 1], (0, pad)).reshape(_ROWS, _LANE)
    tp = jnp.pad(theta[:, 0], (0, pad)).reshape(_ROWS, _LANE)

    ftab = _pre_call(x0, x1).reshape(4 * _NP)
    edges = edge_index.reshape(2 * _E)
    zeros = jnp.zeros((2 * _NP,), jnp.float32)

    part = _sc_call(ftab, edges, zeros)
    pp = part.reshape(_NC, 2, _ROWS, _LANE)

    v0, v1, w = _epi_call(pp, tp)
    v = jnp.stack([v0.reshape(-1)[:_N], v1.reshape(-1)[:_N]], axis=-1)
    wq = w.reshape(-1)[:_N][:, None]
    return (v, wq)
